# baseline (device time: 302293 ns/iter reference)
import jax
import jax.numpy as jnp
from jax import lax
from jax.experimental import pallas as pl
from jax.experimental.pallas import tpu as pltpu

N_PLANE = 8
N_Z = 4
M = 4096
N_COL = 2048
QC = 512
PCH = M // N_PLANE
SUB = PCH // N_Z
RCH = M // N_Z
N_SUP = 20

C_Q1R, C_Q1L = 0, 1024
C_Q2R, C_Q2L = 512, 1536


def kernel(x, w_mat, scale_x, scale_w):
    xb = x.astype(jnp.bfloat16)
    wb = w_mat.astype(jnp.bfloat16)
    sp = (scale_x.astype(jnp.float32) * scale_w.astype(jnp.float32)).reshape(1, 1)

    def body(x_ref, w_ref, sp_ref, out_ref,
             stA_R, stA_L, cmA_R, cmA_L, stB_R, stB_L, cmB_R, cmB_L,
             planeR, planeL, finalR, finalL,
             stZ_R, stZ_L, cmZ_R, cmZ_L, stP_R, stP_L, cmP_R, cmP_L,
             zplaneR, zplaneL, finalZR, finalZL,
             s_q1p_R, r_q1p_R, s_q1p_L, r_q1p_L,
             s_q1z_R, r_q1z_R, s_q1z_L, r_q1z_L,
             s_q2z_R, r_q2z_R, s_q2z_L, r_q2z_L,
             s_q2p_R, r_q2p_R, s_q2p_L, r_q2p_L,
             cr_q1p_R, cr_q1p_L, cr_q1z_R, cr_q1z_L,
             cr_q2z_R, cr_q2z_L, cr_q2p_R, cr_q2p_L):
        d = lax.axis_index("i")
        z = lax.div(d, N_PLANE)
        s_idx = lax.rem(d, N_PLANE)
        y_me = lax.div(s_idx, 2)
        x_me = lax.rem(s_idx + y_me, 2)
        c = jnp.where(x_me == 1, 1 + y_me, lax.rem(8 - y_me, 8))

        def plane_pos(cc):
            xx = jnp.where(cc == 0, 0, jnp.where(cc <= 4, 1, 0))
            yy = jnp.where(cc == 0, 0, jnp.where(cc <= 4, cc - 1, 8 - cc))
            ss = 2 * yy + lax.rem(xx + yy, 2)
            return z * N_PLANE + ss

        qr = plane_pos(lax.rem(c + 1, N_PLANE))
        ql = plane_pos(lax.rem(c + N_PLANE - 1, N_PLANE))
        zr = lax.rem(z + 1, N_Z) * N_PLANE + s_idx
        zl = lax.rem(z + N_Z - 1, N_Z) * N_PLANE + s_idx

        barrier = pltpu.get_barrier_semaphore()
        for nbr in (ql, qr):
            pl.semaphore_signal(barrier, inc=1, device_id=(nbr,),
                                device_id_type=pl.DeviceIdType.MESH)
        pl.semaphore_wait(barrier, 2)

        for sem, tgt in ((cr_q1z_R, zl), (cr_q1z_L, zr),
                         (cr_q2z_R, zl), (cr_q2z_L, zr)):
            pl.semaphore_signal(sem, inc=2, device_id=(tgt,),
                                device_id_type=pl.DeviceIdType.MESH)

        scale = sp_ref[0, 0]

        def dot_part(row0, nrows, col0):
            xa = x_ref[pl.ds(row0, nrows), :]
            wa = w_ref[:, col0:col0 + QC]
            return lax.dot_general(
                xa, wa,
                dimension_numbers=(((1,), (0,)), ((), ())),
                preferred_element_type=jnp.float32)

        def silu(acc):
            yv = acc * scale
            return yv / (1.0 + jnp.exp(-jnp.clip(yv, -60.0, 60.0)))

        def start_pair(stR, cmR, stL, cmL, ssR, rsR, ssL, rsL,
                       slotR, slotL, tR, tL):
            rr = pltpu.make_async_remote_copy(
                src_ref=stR.at[slotR], dst_ref=cmR.at[slotR],
                send_sem=ssR.at[slotR], recv_sem=rsR.at[slotR],
                device_id=(tR,), device_id_type=pl.DeviceIdType.MESH)
            rl = pltpu.make_async_remote_copy(
                src_ref=stL.at[slotL], dst_ref=cmL.at[slotL],
                send_sem=ssL.at[slotL], recv_sem=rsL.at[slotL],
                device_id=(tL,), device_id_type=pl.DeviceIdType.MESH)
            rr.start()
            rl.start()
            return rr, rl

        def sig(sem, tgt, n=1):
            pl.semaphore_signal(sem, inc=n, device_id=(tgt,),
                                device_id_type=pl.DeviceIdType.MESH)

        mod8 = lambda v: lax.rem(v + 2 * N_PLANE, N_PLANE)
        mod4 = lambda v: lax.rem(v + 2 * N_Z, N_Z)

        rows_R = mod8(c + 1) * PCH
        rows_L = mod8(c - 1) * PCH
        rz_R = mod4(z + 1)
        rz_L = mod4(z - 1)

        stA_R[0, :, :] = dot_part(c * PCH, PCH, C_Q1R).astype(jnp.bfloat16)
        stA_L[0, :, :] = dot_part(c * PCH, PCH, C_Q1L).astype(jnp.bfloat16)
        stZ_R[0, :, :] = dot_part(z * RCH, RCH, C_Q2R).astype(jnp.bfloat16)
        stZ_L[0, :, :] = dot_part(z * RCH, RCH, C_Q2L).astype(jnp.bfloat16)

        for i in range(N_SUP):
            q1_phase = 0 if i < 7 else (1 if i < 13 else 2)
            q2_phase = 0 if i < 3 else (1 if i < 17 else 2)
            h1 = i if q1_phase == 0 else (i - 7 if q1_phase == 1 else i - 13)
            h2 = i if q2_phase == 0 else (i - 3 if q2_phase == 1 else i - 17)
            sl1, nx1 = h1 % 2, (h1 + 1) % 2
            sl2, nx2 = h2 % 2, (h2 + 1) % 2

            if i == 7:
                stB_R[0, :, :] = planeR[pl.ds(z * SUB, SUB), :]
                stB_L[0, :, :] = planeL[pl.ds(z * SUB, SUB), :]
            if i == 13:
                stA_R[0, :, :] = finalR[:, :]
                stA_L[0, :, :] = finalL[:, :]
            if i == 3:
                stP_R[0, :, :] = zplaneR[pl.ds(c * SUB, SUB), :]
                stP_L[0, :, :] = zplaneL[pl.ds(c * SUB, SUB), :]
            if i == 17:
                stZ_R[0, :, :] = finalZR[:, :]
                stZ_L[0, :, :] = finalZL[:, :]

            if q1_phase == 0:
                if h1 >= 2:
                    pl.semaphore_wait(cr_q1p_R, 1)
                    pl.semaphore_wait(cr_q1p_L, 1)
            elif q1_phase == 1:
                pl.semaphore_wait(cr_q1z_R, 1)
                pl.semaphore_wait(cr_q1z_L, 1)
            else:
                if h1 == 0:
                    pl.semaphore_wait(cr_q1p_R, 2)
                    pl.semaphore_wait(cr_q1p_L, 2)
                elif h1 >= 2:
                    pl.semaphore_wait(cr_q1p_R, 1)
                    pl.semaphore_wait(cr_q1p_L, 1)
            if q2_phase == 0:
                pl.semaphore_wait(cr_q2z_R, 1)
                pl.semaphore_wait(cr_q2z_L, 1)
            elif q2_phase == 1:
                if h2 >= 2:
                    pl.semaphore_wait(cr_q2p_R, 1)
                    pl.semaphore_wait(cr_q2p_L, 1)
            else:
                if h2 == 0:
                    pl.semaphore_wait(cr_q2z_R, 2)
                    pl.semaphore_wait(cr_q2z_L, 2)
                else:
                    pl.semaphore_wait(cr_q2z_R, 1)
                    pl.semaphore_wait(cr_q2z_L, 1)

            if q1_phase == 1:
                r1, l1 = start_pair(stB_R, cmB_R, stB_L, cmB_L,
                                    s_q1z_R, r_q1z_R, s_q1z_L, r_q1z_L,
                                    sl1, sl1, zr, zl)
            else:
                r1, l1 = start_pair(stA_R, cmA_R, stA_L, cmA_L,
                                    s_q1p_R, r_q1p_R, s_q1p_L, r_q1p_L,
                                    sl1, sl1, qr, ql)
            if q2_phase == 1:
                r2, l2 = start_pair(stP_R, cmP_R, stP_L, cmP_L,
                                    s_q2p_R, r_q2p_R, s_q2p_L, r_q2p_L,
                                    sl2, sl2, qr, ql)
            else:
                r2, l2 = start_pair(stZ_R, cmZ_R, stZ_L, cmZ_L,
                                    s_q2z_R, r_q2z_R, s_q2z_L, r_q2z_L,
                                    sl2, sl2, zr, zl)

            pcs = {}
            if q1_phase == 0 and h1 < 7:
                jR = mod8(c - h1 - 1)
                jL = mod8(c + h1 + 1)
                pcs["q1R"] = dot_part(jR * PCH, PCH, C_Q1R)
                pcs["q1L"] = dot_part(jL * PCH, PCH, C_Q1L)
            if q2_phase == 0 and h2 < 3:
                rR = mod4(z - h2 - 1)
                rL = mod4(z + h2 + 1)
                pcs["q2R"] = dot_part(rR * RCH, RCH, C_Q2R)
                pcs["q2L"] = dot_part(rL * RCH, RCH, C_Q2L)

            r1.wait_send()
            l1.wait_send()
            r2.wait_send()
            l2.wait_send()
            r1.wait_recv()
            l1.wait_recv()
            r2.wait_recv()
            l2.wait_recv()

            if q1_phase == 0:
                accR = cmA_R[sl1, :, :].astype(jnp.float32) + pcs["q1R"]
                accL = cmA_L[sl1, :, :].astype(jnp.float32) + pcs["q1L"]
                if h1 < 6:
                    stA_R[nx1, :, :] = accR.astype(jnp.bfloat16)
                    stA_L[nx1, :, :] = accL.astype(jnp.bfloat16)
                else:
                    planeR[:, :] = accR.astype(jnp.bfloat16)
                    planeL[:, :] = accL.astype(jnp.bfloat16)
                sig(cr_q1p_R, ql)
                sig(cr_q1p_L, qr)
            elif q1_phase == 1:
                if h1 < 3:
                    mR = mod4(z - h1 - 1)
                    mL = mod4(z + h1 + 1)
                    accR = (cmB_R[sl1, :, :].astype(jnp.float32)
                            + planeR[pl.ds(mR * SUB, SUB), :].astype(jnp.float32))
                    accL = (cmB_L[sl1, :, :].astype(jnp.float32)
                            + planeL[pl.ds(mL * SUB, SUB), :].astype(jnp.float32))
                    if h1 == 2:
                        yR = silu(accR)
                        yL = silu(accL)
                        out_ref[pl.ds(rows_R + mR * SUB, SUB),
                                C_Q1R:C_Q1R + QC] = yR
                        out_ref[pl.ds(rows_L + mL * SUB, SUB),
                                C_Q1L:C_Q1L + QC] = yL
                        finalR[pl.ds(mR * SUB, SUB), :] = yR.astype(jnp.bfloat16)
                        finalL[pl.ds(mL * SUB, SUB), :] = yL.astype(jnp.bfloat16)
                        stB_R[nx1, :, :] = yR.astype(jnp.bfloat16)
                        stB_L[nx1, :, :] = yL.astype(jnp.bfloat16)
                    else:
                        stB_R[nx1, :, :] = accR.astype(jnp.bfloat16)
                        stB_L[nx1, :, :] = accL.astype(jnp.bfloat16)
                else:
                    t = h1 - 3
                    mR = mod4(z - t)
                    mL = mod4(z + t)
                    finalR[pl.ds(mR * SUB, SUB), :] = cmB_R[sl1, :, :]
                    finalL[pl.ds(mL * SUB, SUB), :] = cmB_L[sl1, :, :]
                    out_ref[pl.ds(rows_R + mR * SUB, SUB), C_Q1R:C_Q1R + QC] = (
                        cmB_R[sl1, :, :].astype(jnp.float32))
                    out_ref[pl.ds(rows_L + mL * SUB, SUB), C_Q1L:C_Q1L + QC] = (
                        cmB_L[sl1, :, :].astype(jnp.float32))
                    if h1 < 5:
                        stB_R[nx1, :, :] = cmB_R[sl1, :, :]
                        stB_L[nx1, :, :] = cmB_L[sl1, :, :]
                if h1 <= 3:
                    sig(cr_q1z_R, zl)
                    sig(cr_q1z_L, zr)
            else:
                t = h1
                rowsRr = mod8(c - t) * PCH
                rowsLr = mod8(c + t) * PCH
                out_ref[pl.ds(rowsRr, PCH), C_Q1R:C_Q1R + QC] = (
                    cmA_R[sl1, :, :].astype(jnp.float32))
                out_ref[pl.ds(rowsLr, PCH), C_Q1L:C_Q1L + QC] = (
                    cmA_L[sl1, :, :].astype(jnp.float32))
                if t < 6:
                    stA_R[nx1, :, :] = cmA_R[sl1, :, :]
                    stA_L[nx1, :, :] = cmA_L[sl1, :, :]
                if t <= 4:
                    sig(cr_q1p_R, ql)
                    sig(cr_q1p_L, qr)

            if q2_phase == 0:
                accR = cmZ_R[sl2, :, :].astype(jnp.float32) + pcs["q2R"]
                accL = cmZ_L[sl2, :, :].astype(jnp.float32) + pcs["q2L"]
                if h2 < 2:
                    stZ_R[nx2, :, :] = accR.astype(jnp.bfloat16)
                    stZ_L[nx2, :, :] = accL.astype(jnp.bfloat16)
                else:
                    zplaneR[:, :] = accR.astype(jnp.bfloat16)
                    zplaneL[:, :] = accL.astype(jnp.bfloat16)
                sig(cr_q2z_R, zl)
                sig(cr_q2z_L, zr)
            elif q2_phase == 1:
                if h2 < 7:
                    jR = mod8(c - h2 - 1)
                    jL = mod8(c + h2 + 1)
                    accR = (cmP_R[sl2, :, :].astype(jnp.float32)
                            + zplaneR[pl.ds(jR * SUB, SUB), :].astype(jnp.float32))
                    accL = (cmP_L[sl2, :, :].astype(jnp.float32)
                            + zplaneL[pl.ds(jL * SUB, SUB), :].astype(jnp.float32))
                    if h2 == 6:
                        yR = silu(accR)
                        yL = silu(accL)
                        out_ref[pl.ds(rz_R * RCH + jR * SUB, SUB),
                                C_Q2R:C_Q2R + QC] = yR
                        out_ref[pl.ds(rz_L * RCH + jL * SUB, SUB),
                                C_Q2L:C_Q2L + QC] = yL
                        finalZR[pl.ds(jR * SUB, SUB), :] = yR.astype(jnp.bfloat16)
                        finalZL[pl.ds(jL * SUB, SUB), :] = yL.astype(jnp.bfloat16)
                        stP_R[nx2, :, :] = yR.astype(jnp.bfloat16)
                        stP_L[nx2, :, :] = yL.astype(jnp.bfloat16)
                    else:
                        stP_R[nx2, :, :] = accR.astype(jnp.bfloat16)
                        stP_L[nx2, :, :] = accL.astype(jnp.bfloat16)
                else:
                    t = h2 - 7
                    jR = mod8(c - t)
                    jL = mod8(c + t)
                    finalZR[pl.ds(jR * SUB, SUB), :] = cmP_R[sl2, :, :]
                    finalZL[pl.ds(jL * SUB, SUB), :] = cmP_L[sl2, :, :]
                    out_ref[pl.ds(rz_R * RCH + jR * SUB, SUB),
                            C_Q2R:C_Q2R + QC] = (
                        cmP_R[sl2, :, :].astype(jnp.float32))
                    out_ref[pl.ds(rz_L * RCH + jL * SUB, SUB),
                            C_Q2L:C_Q2L + QC] = (
                        cmP_L[sl2, :, :].astype(jnp.float32))
                    if h2 < 13:
                        stP_R[nx2, :, :] = cmP_R[sl2, :, :]
                        stP_L[nx2, :, :] = cmP_L[sl2, :, :]
                if h2 <= 11:
                    sig(cr_q2p_R, ql)
                    sig(cr_q2p_L, qr)
            else:
                t = h2
                rR = mod4(z - t)
                rL = mod4(z + t)
                out_ref[pl.ds(rR * RCH, RCH), C_Q2R:C_Q2R + QC] = (
                    cmZ_R[sl2, :, :].astype(jnp.float32))
                out_ref[pl.ds(rL * RCH, RCH), C_Q2L:C_Q2L + QC] = (
                    cmZ_L[sl2, :, :].astype(jnp.float32))
                if t < 2:
                    stZ_R[nx2, :, :] = cmZ_R[sl2, :, :]
                    stZ_L[nx2, :, :] = cmZ_L[sl2, :, :]
                if t <= 1:
                    sig(cr_q2z_R, zl)
                    sig(cr_q2z_L, zr)

    return pl.pallas_call(
        body,
        out_shape=jax.ShapeDtypeStruct((M, N_COL), jnp.float32),
        in_specs=[
            pl.BlockSpec(memory_space=pltpu.VMEM),
            pl.BlockSpec(memory_space=pltpu.VMEM),
            pl.BlockSpec(memory_space=pltpu.SMEM),
        ],
        out_specs=pl.BlockSpec(memory_space=pltpu.VMEM),
        scratch_shapes=[
            pltpu.VMEM((2, PCH, QC), jnp.bfloat16),
            pltpu.VMEM((2, PCH, QC), jnp.bfloat16),
            pltpu.VMEM((2, PCH, QC), jnp.bfloat16),
            pltpu.VMEM((2, PCH, QC), jnp.bfloat16),
            pltpu.VMEM((2, SUB, QC), jnp.bfloat16),
            pltpu.VMEM((2, SUB, QC), jnp.bfloat16),
            pltpu.VMEM((2, SUB, QC), jnp.bfloat16),
            pltpu.VMEM((2, SUB, QC), jnp.bfloat16),
            pltpu.VMEM((PCH, QC), jnp.bfloat16),
            pltpu.VMEM((PCH, QC), jnp.bfloat16),
            pltpu.VMEM((PCH, QC), jnp.bfloat16),
            pltpu.VMEM((PCH, QC), jnp.bfloat16),
            pltpu.VMEM((2, RCH, QC), jnp.bfloat16),
            pltpu.VMEM((2, RCH, QC), jnp.bfloat16),
            pltpu.VMEM((2, RCH, QC), jnp.bfloat16),
            pltpu.VMEM((2, RCH, QC), jnp.bfloat16),
            pltpu.VMEM((2, SUB, QC), jnp.bfloat16),
            pltpu.VMEM((2, SUB, QC), jnp.bfloat16),
            pltpu.VMEM((2, SUB, QC), jnp.bfloat16),
            pltpu.VMEM((2, SUB, QC), jnp.bfloat16),
            pltpu.VMEM((RCH, QC), jnp.bfloat16),
            pltpu.VMEM((RCH, QC), jnp.bfloat16),
            pltpu.VMEM((RCH, QC), jnp.bfloat16),
            pltpu.VMEM((RCH, QC), jnp.bfloat16),
            pltpu.SemaphoreType.DMA((2,)),
            pltpu.SemaphoreType.DMA((2,)),
            pltpu.SemaphoreType.DMA((2,)),
            pltpu.SemaphoreType.DMA((2,)),
            pltpu.SemaphoreType.DMA((2,)),
            pltpu.SemaphoreType.DMA((2,)),
            pltpu.SemaphoreType.DMA((2,)),
            pltpu.SemaphoreType.DMA((2,)),
            pltpu.SemaphoreType.DMA((2,)),
            pltpu.SemaphoreType.DMA((2,)),
            pltpu.SemaphoreType.DMA((2,)),
            pltpu.SemaphoreType.DMA((2,)),
            pltpu.SemaphoreType.DMA((2,)),
            pltpu.SemaphoreType.DMA((2,)),
            pltpu.SemaphoreType.DMA((2,)),
            pltpu.SemaphoreType.DMA((2,)),
            pltpu.SemaphoreType.REGULAR,
            pltpu.SemaphoreType.REGULAR,
            pltpu.SemaphoreType.REGULAR,
            pltpu.SemaphoreType.REGULAR,
            pltpu.SemaphoreType.REGULAR,
            pltpu.SemaphoreType.REGULAR,
            pltpu.SemaphoreType.REGULAR,
            pltpu.SemaphoreType.REGULAR,
        ],
        compiler_params=pltpu.CompilerParams(
            collective_id=0, vmem_limit_bytes=64 * 1024 * 1024),
    )(xb, wb, sp)


# device time: 301183 ns/iter; 1.0037x vs baseline; 1.0037x over previous
import jax
import jax.numpy as jnp
from jax import lax
from jax.experimental import pallas as pl
from jax.experimental.pallas import tpu as pltpu

N_PLANE = 8
N_Z = 4
M = 4096
N_COL = 2048
QC = 512
PCH = M // N_PLANE
SUB = PCH // N_Z
RCH = M // N_Z
N_SUP = 20

C_Q1R, C_Q1L = 0, 1024
C_Q2R, C_Q2L = 512, 1536


def kernel(x, w_mat, scale_x, scale_w):
    xb = x.astype(jnp.bfloat16)
    wb = w_mat.astype(jnp.bfloat16)
    sp = (scale_x.astype(jnp.float32) * scale_w.astype(jnp.float32)).reshape(1, 1)

    def body(x_ref, w_ref, sp_ref, out_ref,
             stA_R, stA_L, cmA_R, cmA_L, stB_R, stB_L, cmB_R, cmB_L,
             planeR, planeL, finalR, finalL,
             stZ_R, stZ_L, cmZ_R, cmZ_L, stP_R, stP_L, cmP_R, cmP_L,
             zplaneR, zplaneL, finalZR, finalZL,
             s_q1p_R, r_q1p_R, s_q1p_L, r_q1p_L,
             s_q1z_R, r_q1z_R, s_q1z_L, r_q1z_L,
             s_q2z_R, r_q2z_R, s_q2z_L, r_q2z_L,
             s_q2p_R, r_q2p_R, s_q2p_L, r_q2p_L,
             cr_q1p_R, cr_q1p_L, cr_q1z_R, cr_q1z_L,
             cr_q2z_R, cr_q2z_L, cr_q2p_R, cr_q2p_L):
        d = lax.axis_index("i")
        z = lax.div(d, N_PLANE)
        s_idx = lax.rem(d, N_PLANE)
        y_me = lax.div(s_idx, 2)
        x_me = lax.rem(s_idx + y_me, 2)
        c = jnp.where(x_me == 1, 1 + y_me, lax.rem(8 - y_me, 8))

        def plane_pos(cc):
            xx = jnp.where(cc == 0, 0, jnp.where(cc <= 4, 1, 0))
            yy = jnp.where(cc == 0, 0, jnp.where(cc <= 4, cc - 1, 8 - cc))
            ss = 2 * yy + lax.rem(xx + yy, 2)
            return z * N_PLANE + ss

        qr = plane_pos(lax.rem(c + 1, N_PLANE))
        ql = plane_pos(lax.rem(c + N_PLANE - 1, N_PLANE))
        zr = lax.rem(z + 1, N_Z) * N_PLANE + s_idx
        zl = lax.rem(z + N_Z - 1, N_Z) * N_PLANE + s_idx

        barrier = pltpu.get_barrier_semaphore()
        for nbr in (ql, qr):
            pl.semaphore_signal(barrier, inc=1, device_id=(nbr,),
                                device_id_type=pl.DeviceIdType.MESH)
        pl.semaphore_wait(barrier, 2)

        for sem, tgt in ((cr_q1z_R, zl), (cr_q1z_L, zr),
                         (cr_q2z_R, zl), (cr_q2z_L, zr)):
            pl.semaphore_signal(sem, inc=2, device_id=(tgt,),
                                device_id_type=pl.DeviceIdType.MESH)

        scale = sp_ref[0, 0]

        def dot_part(row0, nrows, col0):
            xa = x_ref[pl.ds(row0, nrows), :]
            wa = w_ref[:, col0:col0 + QC]
            return lax.dot_general(
                xa, wa,
                dimension_numbers=(((1,), (0,)), ((), ())),
                preferred_element_type=jnp.float32)

        def silu(acc):
            yv = acc * scale
            return yv / (1.0 + jnp.exp(-jnp.clip(yv, -60.0, 60.0)))

        def start_pair(stR, cmR, stL, cmL, ssR, rsR, ssL, rsL,
                       slotR, slotL, tR, tL):
            rr = pltpu.make_async_remote_copy(
                src_ref=stR.at[slotR], dst_ref=cmR.at[slotR],
                send_sem=ssR.at[slotR], recv_sem=rsR.at[slotR],
                device_id=(tR,), device_id_type=pl.DeviceIdType.MESH)
            rl = pltpu.make_async_remote_copy(
                src_ref=stL.at[slotL], dst_ref=cmL.at[slotL],
                send_sem=ssL.at[slotL], recv_sem=rsL.at[slotL],
                device_id=(tL,), device_id_type=pl.DeviceIdType.MESH)
            rr.start()
            rl.start()
            return rr, rl

        def sig(sem, tgt, n=1):
            pl.semaphore_signal(sem, inc=n, device_id=(tgt,),
                                device_id_type=pl.DeviceIdType.MESH)

        mod8 = lambda v: lax.rem(v + 2 * N_PLANE, N_PLANE)
        mod4 = lambda v: lax.rem(v + 2 * N_Z, N_Z)

        rows_R = mod8(c + 1) * PCH
        rows_L = mod8(c - 1) * PCH
        rz_R = mod4(z + 1)
        rz_L = mod4(z - 1)

        stA_R[0, :, :] = dot_part(c * PCH, PCH, C_Q1R).astype(jnp.bfloat16)
        stA_L[0, :, :] = dot_part(c * PCH, PCH, C_Q1L).astype(jnp.bfloat16)
        stZ_R[0, :, :] = dot_part(z * RCH, RCH, C_Q2R).astype(jnp.bfloat16)
        stZ_L[0, :, :] = dot_part(z * RCH, RCH, C_Q2L).astype(jnp.bfloat16)

        for i in range(N_SUP):
            q1_phase = 0 if i < 7 else (1 if i < 13 else 2)
            q2_phase = 0 if i < 3 else (1 if i < 17 else 2)
            h1 = i if q1_phase == 0 else (i - 7 if q1_phase == 1 else i - 13)
            h2 = i if q2_phase == 0 else (i - 3 if q2_phase == 1 else i - 17)
            sl1, nx1 = h1 % 2, (h1 + 1) % 2
            sl2, nx2 = h2 % 2, (h2 + 1) % 2

            if i == 7:
                stB_R[0, :, :] = planeR[pl.ds(z * SUB, SUB), :]
                stB_L[0, :, :] = planeL[pl.ds(z * SUB, SUB), :]
            if i == 13:
                stA_R[0, :, :] = finalR[:, :]
                stA_L[0, :, :] = finalL[:, :]
            if i == 3:
                stP_R[0, :, :] = zplaneR[pl.ds(c * SUB, SUB), :]
                stP_L[0, :, :] = zplaneL[pl.ds(c * SUB, SUB), :]
            if i == 17:
                stZ_R[0, :, :] = finalZR[:, :]
                stZ_L[0, :, :] = finalZL[:, :]

            if q1_phase == 0:
                if h1 >= 2:
                    pl.semaphore_wait(cr_q1p_R, 1)
                    pl.semaphore_wait(cr_q1p_L, 1)
            elif q1_phase == 1:
                pl.semaphore_wait(cr_q1z_R, 1)
                pl.semaphore_wait(cr_q1z_L, 1)
            else:
                if h1 == 0:
                    pl.semaphore_wait(cr_q1p_R, 2)
                    pl.semaphore_wait(cr_q1p_L, 2)
                elif h1 >= 2:
                    pl.semaphore_wait(cr_q1p_R, 1)
                    pl.semaphore_wait(cr_q1p_L, 1)
            if q2_phase == 0:
                pl.semaphore_wait(cr_q2z_R, 1)
                pl.semaphore_wait(cr_q2z_L, 1)
            elif q2_phase == 1:
                if h2 >= 2:
                    pl.semaphore_wait(cr_q2p_R, 1)
                    pl.semaphore_wait(cr_q2p_L, 1)
            else:
                if h2 == 0:
                    pl.semaphore_wait(cr_q2z_R, 2)
                    pl.semaphore_wait(cr_q2z_L, 2)
                else:
                    pl.semaphore_wait(cr_q2z_R, 1)
                    pl.semaphore_wait(cr_q2z_L, 1)

            if q1_phase == 1:
                r1, l1 = start_pair(stB_R, cmB_R, stB_L, cmB_L,
                                    s_q1z_R, r_q1z_R, s_q1z_L, r_q1z_L,
                                    sl1, sl1, zr, zl)
            else:
                r1, l1 = start_pair(stA_R, cmA_R, stA_L, cmA_L,
                                    s_q1p_R, r_q1p_R, s_q1p_L, r_q1p_L,
                                    sl1, sl1, qr, ql)
            if q2_phase == 1:
                r2, l2 = start_pair(stP_R, cmP_R, stP_L, cmP_L,
                                    s_q2p_R, r_q2p_R, s_q2p_L, r_q2p_L,
                                    sl2, sl2, qr, ql)
            else:
                r2, l2 = start_pair(stZ_R, cmZ_R, stZ_L, cmZ_L,
                                    s_q2z_R, r_q2z_R, s_q2z_L, r_q2z_L,
                                    sl2, sl2, zr, zl)

            pcs = {}
            if q1_phase == 0 and h1 < 7:
                jR = mod8(c - h1 - 1)
                jL = mod8(c + h1 + 1)
                pcs["q1R"] = dot_part(jR * PCH, PCH, C_Q1R)
                pcs["q1L"] = dot_part(jL * PCH, PCH, C_Q1L)
            if q2_phase == 0 and h2 < 3:
                rR = mod4(z - h2 - 1)
                rL = mod4(z + h2 + 1)
                pcs["q2R"] = dot_part(rR * RCH, RCH, C_Q2R)
                pcs["q2L"] = dot_part(rL * RCH, RCH, C_Q2L)

            def q1_consume():
                if q1_phase == 0:
                    accR = cmA_R[sl1, :, :].astype(jnp.float32) + pcs["q1R"]
                    accL = cmA_L[sl1, :, :].astype(jnp.float32) + pcs["q1L"]
                    if h1 < 6:
                        stA_R[nx1, :, :] = accR.astype(jnp.bfloat16)
                        stA_L[nx1, :, :] = accL.astype(jnp.bfloat16)
                    else:
                        planeR[:, :] = accR.astype(jnp.bfloat16)
                        planeL[:, :] = accL.astype(jnp.bfloat16)
                    sig(cr_q1p_R, ql)
                    sig(cr_q1p_L, qr)
                elif q1_phase == 1:
                    if h1 < 3:
                        mR = mod4(z - h1 - 1)
                        mL = mod4(z + h1 + 1)
                        accR = (cmB_R[sl1, :, :].astype(jnp.float32)
                                + planeR[pl.ds(mR * SUB, SUB), :].astype(jnp.float32))
                        accL = (cmB_L[sl1, :, :].astype(jnp.float32)
                                + planeL[pl.ds(mL * SUB, SUB), :].astype(jnp.float32))
                        if h1 == 2:
                            yR = silu(accR)
                            yL = silu(accL)
                            out_ref[pl.ds(rows_R + mR * SUB, SUB),
                                    C_Q1R:C_Q1R + QC] = yR
                            out_ref[pl.ds(rows_L + mL * SUB, SUB),
                                    C_Q1L:C_Q1L + QC] = yL
                            finalR[pl.ds(mR * SUB, SUB), :] = yR.astype(jnp.bfloat16)
                            finalL[pl.ds(mL * SUB, SUB), :] = yL.astype(jnp.bfloat16)
                            stB_R[nx1, :, :] = yR.astype(jnp.bfloat16)
                            stB_L[nx1, :, :] = yL.astype(jnp.bfloat16)
                        else:
                            stB_R[nx1, :, :] = accR.astype(jnp.bfloat16)
                            stB_L[nx1, :, :] = accL.astype(jnp.bfloat16)
                    else:
                        t = h1 - 3
                        mR = mod4(z - t)
                        mL = mod4(z + t)
                        finalR[pl.ds(mR * SUB, SUB), :] = cmB_R[sl1, :, :]
                        finalL[pl.ds(mL * SUB, SUB), :] = cmB_L[sl1, :, :]
                        out_ref[pl.ds(rows_R + mR * SUB, SUB), C_Q1R:C_Q1R + QC] = (
                            cmB_R[sl1, :, :].astype(jnp.float32))
                        out_ref[pl.ds(rows_L + mL * SUB, SUB), C_Q1L:C_Q1L + QC] = (
                            cmB_L[sl1, :, :].astype(jnp.float32))
                        if h1 < 5:
                            stB_R[nx1, :, :] = cmB_R[sl1, :, :]
                            stB_L[nx1, :, :] = cmB_L[sl1, :, :]
                    if h1 <= 3:
                        sig(cr_q1z_R, zl)
                        sig(cr_q1z_L, zr)
                else:
                    t = h1
                    rowsRr = mod8(c - t) * PCH
                    rowsLr = mod8(c + t) * PCH
                    out_ref[pl.ds(rowsRr, PCH), C_Q1R:C_Q1R + QC] = (
                        cmA_R[sl1, :, :].astype(jnp.float32))
                    out_ref[pl.ds(rowsLr, PCH), C_Q1L:C_Q1L + QC] = (
                        cmA_L[sl1, :, :].astype(jnp.float32))
                    if t < 6:
                        stA_R[nx1, :, :] = cmA_R[sl1, :, :]
                        stA_L[nx1, :, :] = cmA_L[sl1, :, :]
                    if t <= 4:
                        sig(cr_q1p_R, ql)
                        sig(cr_q1p_L, qr)

            def q2_consume():
                if q2_phase == 0:
                    accR = cmZ_R[sl2, :, :].astype(jnp.float32) + pcs["q2R"]
                    accL = cmZ_L[sl2, :, :].astype(jnp.float32) + pcs["q2L"]
                    if h2 < 2:
                        stZ_R[nx2, :, :] = accR.astype(jnp.bfloat16)
                        stZ_L[nx2, :, :] = accL.astype(jnp.bfloat16)
                    else:
                        zplaneR[:, :] = accR.astype(jnp.bfloat16)
                        zplaneL[:, :] = accL.astype(jnp.bfloat16)
                    sig(cr_q2z_R, zl)
                    sig(cr_q2z_L, zr)
                elif q2_phase == 1:
                    if h2 < 7:
                        jR = mod8(c - h2 - 1)
                        jL = mod8(c + h2 + 1)
                        accR = (cmP_R[sl2, :, :].astype(jnp.float32)
                                + zplaneR[pl.ds(jR * SUB, SUB), :].astype(jnp.float32))
                        accL = (cmP_L[sl2, :, :].astype(jnp.float32)
                                + zplaneL[pl.ds(jL * SUB, SUB), :].astype(jnp.float32))
                        if h2 == 6:
                            yR = silu(accR)
                            yL = silu(accL)
                            out_ref[pl.ds(rz_R * RCH + jR * SUB, SUB),
                                    C_Q2R:C_Q2R + QC] = yR
                            out_ref[pl.ds(rz_L * RCH + jL * SUB, SUB),
                                    C_Q2L:C_Q2L + QC] = yL
                            finalZR[pl.ds(jR * SUB, SUB), :] = yR.astype(jnp.bfloat16)
                            finalZL[pl.ds(jL * SUB, SUB), :] = yL.astype(jnp.bfloat16)
                            stP_R[nx2, :, :] = yR.astype(jnp.bfloat16)
                            stP_L[nx2, :, :] = yL.astype(jnp.bfloat16)
                        else:
                            stP_R[nx2, :, :] = accR.astype(jnp.bfloat16)
                            stP_L[nx2, :, :] = accL.astype(jnp.bfloat16)
                    else:
                        t = h2 - 7
                        jR = mod8(c - t)
                        jL = mod8(c + t)
                        finalZR[pl.ds(jR * SUB, SUB), :] = cmP_R[sl2, :, :]
                        finalZL[pl.ds(jL * SUB, SUB), :] = cmP_L[sl2, :, :]
                        out_ref[pl.ds(rz_R * RCH + jR * SUB, SUB),
                                C_Q2R:C_Q2R + QC] = (
                            cmP_R[sl2, :, :].astype(jnp.float32))
                        out_ref[pl.ds(rz_L * RCH + jL * SUB, SUB),
                                C_Q2L:C_Q2L + QC] = (
                            cmP_L[sl2, :, :].astype(jnp.float32))
                        if h2 < 13:
                            stP_R[nx2, :, :] = cmP_R[sl2, :, :]
                            stP_L[nx2, :, :] = cmP_L[sl2, :, :]
                    if h2 <= 11:
                        sig(cr_q2p_R, ql)
                        sig(cr_q2p_L, qr)
                else:
                    t = h2
                    rR = mod4(z - t)
                    rL = mod4(z + t)
                    out_ref[pl.ds(rR * RCH, RCH), C_Q2R:C_Q2R + QC] = (
                        cmZ_R[sl2, :, :].astype(jnp.float32))
                    out_ref[pl.ds(rL * RCH, RCH), C_Q2L:C_Q2L + QC] = (
                        cmZ_L[sl2, :, :].astype(jnp.float32))
                    if t < 2:
                        stZ_R[nx2, :, :] = cmZ_R[sl2, :, :]
                        stZ_L[nx2, :, :] = cmZ_L[sl2, :, :]
                    if t <= 1:
                        sig(cr_q2z_R, zl)
                        sig(cr_q2z_L, zr)

            def q1_block():
                r1.wait_send()
                l1.wait_send()
                r1.wait_recv()
                l1.wait_recv()
                q1_consume()

            def q2_block():
                r2.wait_send()
                l2.wait_send()
                r2.wait_recv()
                l2.wait_recv()
                q2_consume()

            if q1_phase == 1 or q2_phase != 1:
                q1_block()
                q2_block()
            else:
                q2_block()
                q1_block()

    return pl.pallas_call(
        body,
        out_shape=jax.ShapeDtypeStruct((M, N_COL), jnp.float32),
        in_specs=[
            pl.BlockSpec(memory_space=pltpu.VMEM),
            pl.BlockSpec(memory_space=pltpu.VMEM),
            pl.BlockSpec(memory_space=pltpu.SMEM),
        ],
        out_specs=pl.BlockSpec(memory_space=pltpu.VMEM),
        scratch_shapes=[
            pltpu.VMEM((2, PCH, QC), jnp.bfloat16),
            pltpu.VMEM((2, PCH, QC), jnp.bfloat16),
            pltpu.VMEM((2, PCH, QC), jnp.bfloat16),
            pltpu.VMEM((2, PCH, QC), jnp.bfloat16),
            pltpu.VMEM((2, SUB, QC), jnp.bfloat16),
            pltpu.VMEM((2, SUB, QC), jnp.bfloat16),
            pltpu.VMEM((2, SUB, QC), jnp.bfloat16),
            pltpu.VMEM((2, SUB, QC), jnp.bfloat16),
            pltpu.VMEM((PCH, QC), jnp.bfloat16),
            pltpu.VMEM((PCH, QC), jnp.bfloat16),
            pltpu.VMEM((PCH, QC), jnp.bfloat16),
            pltpu.VMEM((PCH, QC), jnp.bfloat16),
            pltpu.VMEM((2, RCH, QC), jnp.bfloat16),
            pltpu.VMEM((2, RCH, QC), jnp.bfloat16),
            pltpu.VMEM((2, RCH, QC), jnp.bfloat16),
            pltpu.VMEM((2, RCH, QC), jnp.bfloat16),
            pltpu.VMEM((2, SUB, QC), jnp.bfloat16),
            pltpu.VMEM((2, SUB, QC), jnp.bfloat16),
            pltpu.VMEM((2, SUB, QC), jnp.bfloat16),
            pltpu.VMEM((2, SUB, QC), jnp.bfloat16),
            pltpu.VMEM((RCH, QC), jnp.bfloat16),
            pltpu.VMEM((RCH, QC), jnp.bfloat16),
            pltpu.VMEM((RCH, QC), jnp.bfloat16),
            pltpu.VMEM((RCH, QC), jnp.bfloat16),
            pltpu.SemaphoreType.DMA((2,)),
            pltpu.SemaphoreType.DMA((2,)),
            pltpu.SemaphoreType.DMA((2,)),
            pltpu.SemaphoreType.DMA((2,)),
            pltpu.SemaphoreType.DMA((2,)),
            pltpu.SemaphoreType.DMA((2,)),
            pltpu.SemaphoreType.DMA((2,)),
            pltpu.SemaphoreType.DMA((2,)),
            pltpu.SemaphoreType.DMA((2,)),
            pltpu.SemaphoreType.DMA((2,)),
            pltpu.SemaphoreType.DMA((2,)),
            pltpu.SemaphoreType.DMA((2,)),
            pltpu.SemaphoreType.DMA((2,)),
            pltpu.SemaphoreType.DMA((2,)),
            pltpu.SemaphoreType.DMA((2,)),
            pltpu.SemaphoreType.DMA((2,)),
            pltpu.SemaphoreType.REGULAR,
            pltpu.SemaphoreType.REGULAR,
            pltpu.SemaphoreType.REGULAR,
            pltpu.SemaphoreType.REGULAR,
            pltpu.SemaphoreType.REGULAR,
            pltpu.SemaphoreType.REGULAR,
            pltpu.SemaphoreType.REGULAR,
            pltpu.SemaphoreType.REGULAR,
        ],
        compiler_params=pltpu.CompilerParams(
            collective_id=0, vmem_limit_bytes=64 * 1024 * 1024),
    )(xb, wb, sp)


# device time: 276178 ns/iter; 1.0946x vs baseline; 1.0905x over previous
import jax
import jax.numpy as jnp
from jax import lax
from jax.experimental import pallas as pl
from jax.experimental.pallas import tpu as pltpu

N_PLANE = 8
N_Z = 4
M = 4096
N_COL = 2048
HC = N_COL // 2
PCH = M // N_PLANE
SUB = PCH // N_Z
HPC = PCH // 2


def kernel(x, w_mat, scale_x, scale_w):
    xb = x.astype(jnp.bfloat16)
    wb = w_mat.astype(jnp.bfloat16)
    sp = (scale_x.astype(jnp.float32) * scale_w.astype(jnp.float32)).reshape(1, 1)

    def body(x_ref, w_ref, sp_ref, out_ref,
             stageAR, stageAL, commAR, commAL,
             stageBR, stageBL, commBR, commBL,
             planeR, planeL, finalR, finalL,
             sendR, sendL, recvR, recvL,
             sendBR, sendBL, recvBR, recvBL,
             creditR, creditL, creditBR, creditBL):
        d = lax.axis_index("i")
        z = lax.div(d, N_PLANE)
        s_idx = lax.rem(d, N_PLANE)
        y_me = lax.div(s_idx, 2)
        x_me = lax.rem(s_idx + y_me, 2)
        c = jnp.where(x_me == 1, 1 + y_me, lax.rem(8 - y_me, 8))

        def plane_pos(cc):
            xx = jnp.where(cc == 0, 0, jnp.where(cc <= 4, 1, 0))
            yy = jnp.where(cc == 0, 0, jnp.where(cc <= 4, cc - 1, 8 - cc))
            ss = 2 * yy + lax.rem(xx + yy, 2)
            return z * N_PLANE + ss

        qr = plane_pos(lax.rem(c + 1, N_PLANE))
        ql = plane_pos(lax.rem(c + N_PLANE - 1, N_PLANE))
        zr = lax.rem(z + 1, N_Z) * N_PLANE + s_idx
        zl = lax.rem(z + N_Z - 1, N_Z) * N_PLANE + s_idx

        barrier = pltpu.get_barrier_semaphore()
        for nbr in (ql, qr):
            pl.semaphore_signal(barrier, inc=1, device_id=(nbr,),
                                device_id_type=pl.DeviceIdType.MESH)
        pl.semaphore_wait(barrier, 2)

        pl.semaphore_signal(creditBR, inc=2, device_id=(zl,),
                            device_id_type=pl.DeviceIdType.MESH)
        pl.semaphore_signal(creditBL, inc=2, device_id=(zr,),
                            device_id_type=pl.DeviceIdType.MESH)

        scale = sp_ref[0, 0]

        def pchunkA(j, dir_):
            xa = x_ref[pl.ds(j * PCH, PCH), :]
            wa = w_ref[:, dir_ * HC:(dir_ + 1) * HC]
            return lax.dot_general(
                xa, wa,
                dimension_numbers=(((1,), (0,)), ((), ())),
                preferred_element_type=jnp.float32)

        def silu(acc):
            yv = acc * scale
            return yv / (1.0 + jnp.exp(-jnp.clip(yv, -60.0, 60.0)))

        def exchange(stR, cmR, stL, cmL, ssR, rsR, ssL, rsL, slot, tR, tL):
            rr = pltpu.make_async_remote_copy(
                src_ref=stR.at[slot], dst_ref=cmR.at[slot],
                send_sem=ssR.at[slot], recv_sem=rsR.at[slot],
                device_id=(tR,), device_id_type=pl.DeviceIdType.MESH)
            rl = pltpu.make_async_remote_copy(
                src_ref=stL.at[slot], dst_ref=cmL.at[slot],
                send_sem=ssL.at[slot], recv_sem=rsL.at[slot],
                device_id=(tL,), device_id_type=pl.DeviceIdType.MESH)
            rr.start()
            rl.start()
            return rr, rl

        def exchange_subs(slot):
            subs = []
            for u in (0, 1):
                rr = pltpu.make_async_remote_copy(
                    src_ref=stageAR.at[slot, pl.ds(u * HPC, HPC), :],
                    dst_ref=commAR.at[slot, pl.ds(u * HPC, HPC), :],
                    send_sem=sendR.at[slot * 2 + u],
                    recv_sem=recvR.at[slot * 2 + u],
                    device_id=(qr,), device_id_type=pl.DeviceIdType.MESH)
                rl = pltpu.make_async_remote_copy(
                    src_ref=stageAL.at[slot, pl.ds(u * HPC, HPC), :],
                    dst_ref=commAL.at[slot, pl.ds(u * HPC, HPC), :],
                    send_sem=sendL.at[slot * 2 + u],
                    recv_sem=recvL.at[slot * 2 + u],
                    device_id=(ql,), device_id_type=pl.DeviceIdType.MESH)
                subs.append((rr, rl))
            subs[0][0].start()
            subs[0][1].start()
            subs[1][0].start()
            subs[1][1].start()
            return subs

        stageAR[0, :, :] = pchunkA(c, 0).astype(jnp.bfloat16)
        stageAL[0, :, :] = pchunkA(c, 1).astype(jnp.bfloat16)

        for s in range(N_PLANE - 1):
            slot, nxt = s % 2, (s + 1) % 2
            if s >= 2:
                pl.semaphore_wait(creditR, 1)
                pl.semaphore_wait(creditL, 1)
            subs = exchange_subs(slot)
            jR = lax.rem(c - (s + 1) + 2 * N_PLANE, N_PLANE)
            jL = lax.rem(c + (s + 1), N_PLANE)
            pcR = pchunkA(jR, 0)
            pcL = pchunkA(jL, 1)
            for u in (0, 1):
                rr, rl = subs[u]
                rr.wait_send()
                rl.wait_send()
                rr.wait_recv()
                rl.wait_recv()
                rsl = slice(u * HPC, (u + 1) * HPC)
                accR = (commAR[slot, rsl, :].astype(jnp.float32)
                        + pcR[rsl, :])
                accL = (commAL[slot, rsl, :].astype(jnp.float32)
                        + pcL[rsl, :])
                if s < N_PLANE - 2:
                    stageAR[nxt, rsl, :] = accR.astype(jnp.bfloat16)
                    stageAL[nxt, rsl, :] = accL.astype(jnp.bfloat16)
                else:
                    planeR[rsl, :] = accR.astype(jnp.bfloat16)
                    planeL[rsl, :] = accL.astype(jnp.bfloat16)
            pl.semaphore_signal(creditR, inc=1, device_id=(ql,),
                                device_id_type=pl.DeviceIdType.MESH)
            pl.semaphore_signal(creditL, inc=1, device_id=(qr,),
                                device_id_type=pl.DeviceIdType.MESH)

        rows_R = lax.rem(c + 1, N_PLANE) * PCH
        rows_L = lax.rem(c + N_PLANE - 1, N_PLANE) * PCH

        stageBR[0, :, :] = planeR[pl.ds(z * SUB, SUB), :]
        stageBL[0, :, :] = planeL[pl.ds(z * SUB, SUB), :]

        for b in range(2 * N_Z - 2):
            slot, nxt = b % 2, (b + 1) % 2
            pl.semaphore_wait(creditBR, 1)
            pl.semaphore_wait(creditBL, 1)
            rr, rl = exchange(stageBR, commBR, stageBL, commBL,
                              sendBR, recvBR, sendBL, recvBL, slot, zr, zl)
            rr.wait_send()
            rl.wait_send()
            rr.wait_recv()
            rl.wait_recv()
            if b < N_Z - 1:
                mR = lax.rem(z - (b + 1) + 2 * N_Z, N_Z)
                mL = lax.rem(z + (b + 1), N_Z)
                accR = (commBR[slot, :, :].astype(jnp.float32)
                        + planeR[pl.ds(mR * SUB, SUB), :].astype(jnp.float32))
                accL = (commBL[slot, :, :].astype(jnp.float32)
                        + planeL[pl.ds(mL * SUB, SUB), :].astype(jnp.float32))
                if b == N_Z - 2:
                    yR = silu(accR)
                    yL = silu(accL)
                    out_ref[pl.ds(rows_R + mR * SUB, SUB), 0:HC] = yR
                    out_ref[pl.ds(rows_L + mL * SUB, SUB), HC:N_COL] = yL
                    finalR[pl.ds(mR * SUB, SUB), :] = yR.astype(jnp.bfloat16)
                    finalL[pl.ds(mL * SUB, SUB), :] = yL.astype(jnp.bfloat16)
                    stageBR[nxt, :, :] = yR.astype(jnp.bfloat16)
                    stageBL[nxt, :, :] = yL.astype(jnp.bfloat16)
                else:
                    stageBR[nxt, :, :] = accR.astype(jnp.bfloat16)
                    stageBL[nxt, :, :] = accL.astype(jnp.bfloat16)
            else:
                t = b - (N_Z - 1)
                mR = lax.rem(z - t + 2 * N_Z, N_Z)
                mL = lax.rem(z + t, N_Z)
                finalR[pl.ds(mR * SUB, SUB), :] = commBR[slot, :, :]
                finalL[pl.ds(mL * SUB, SUB), :] = commBL[slot, :, :]
                out_ref[pl.ds(rows_R + mR * SUB, SUB), 0:HC] = (
                    commBR[slot, :, :].astype(jnp.float32))
                out_ref[pl.ds(rows_L + mL * SUB, SUB), HC:N_COL] = (
                    commBL[slot, :, :].astype(jnp.float32))
                if b < 2 * N_Z - 3:
                    stageBR[nxt, :, :] = commBR[slot, :, :]
                    stageBL[nxt, :, :] = commBL[slot, :, :]
            if b <= 2 * N_Z - 5:
                pl.semaphore_signal(creditBR, inc=1, device_id=(zl,),
                                    device_id_type=pl.DeviceIdType.MESH)
                pl.semaphore_signal(creditBL, inc=1, device_id=(zr,),
                                    device_id_type=pl.DeviceIdType.MESH)

        stageAR[0, :, :] = finalR[:, :]
        stageAL[0, :, :] = finalL[:, :]

        for t in range(N_PLANE - 1):
            slot, nxt = t % 2, (t + 1) % 2
            if t == 0:
                pl.semaphore_wait(creditR, 2)
                pl.semaphore_wait(creditL, 2)
            elif t >= 2:
                pl.semaphore_wait(creditR, 1)
                pl.semaphore_wait(creditL, 1)
            subs = exchange_subs(slot)
            rowsRr = lax.rem(c - t + 2 * N_PLANE, N_PLANE) * PCH
            rowsLr = lax.rem(c + t, N_PLANE) * PCH
            for u in (0, 1):
                rr, rl = subs[u]
                rr.wait_send()
                rl.wait_send()
                rr.wait_recv()
                rl.wait_recv()
                rsl = slice(u * HPC, (u + 1) * HPC)
                out_ref[pl.ds(rowsRr + u * HPC, HPC), 0:HC] = (
                    commAR[slot, rsl, :].astype(jnp.float32))
                out_ref[pl.ds(rowsLr + u * HPC, HPC), HC:N_COL] = (
                    commAL[slot, rsl, :].astype(jnp.float32))
                if t < N_PLANE - 2:
                    stageAR[nxt, rsl, :] = commAR[slot, rsl, :]
                    stageAL[nxt, rsl, :] = commAL[slot, rsl, :]
            if t <= N_PLANE - 4:
                pl.semaphore_signal(creditR, inc=1, device_id=(ql,),
                                    device_id_type=pl.DeviceIdType.MESH)
                pl.semaphore_signal(creditL, inc=1, device_id=(qr,),
                                    device_id_type=pl.DeviceIdType.MESH)

    return pl.pallas_call(
        body,
        out_shape=jax.ShapeDtypeStruct((M, N_COL), jnp.float32),
        in_specs=[
            pl.BlockSpec(memory_space=pltpu.VMEM),
            pl.BlockSpec(memory_space=pltpu.VMEM),
            pl.BlockSpec(memory_space=pltpu.SMEM),
        ],
        out_specs=pl.BlockSpec(memory_space=pltpu.VMEM),
        scratch_shapes=[
            pltpu.VMEM((2, PCH, HC), jnp.bfloat16),
            pltpu.VMEM((2, PCH, HC), jnp.bfloat16),
            pltpu.VMEM((2, PCH, HC), jnp.bfloat16),
            pltpu.VMEM((2, PCH, HC), jnp.bfloat16),
            pltpu.VMEM((2, SUB, HC), jnp.bfloat16),
            pltpu.VMEM((2, SUB, HC), jnp.bfloat16),
            pltpu.VMEM((2, SUB, HC), jnp.bfloat16),
            pltpu.VMEM((2, SUB, HC), jnp.bfloat16),
            pltpu.VMEM((PCH, HC), jnp.bfloat16),
            pltpu.VMEM((PCH, HC), jnp.bfloat16),
            pltpu.VMEM((PCH, HC), jnp.bfloat16),
            pltpu.VMEM((PCH, HC), jnp.bfloat16),
            pltpu.SemaphoreType.DMA((4,)),
            pltpu.SemaphoreType.DMA((4,)),
            pltpu.SemaphoreType.DMA((4,)),
            pltpu.SemaphoreType.DMA((4,)),
            pltpu.SemaphoreType.DMA((2,)),
            pltpu.SemaphoreType.DMA((2,)),
            pltpu.SemaphoreType.DMA((2,)),
            pltpu.SemaphoreType.DMA((2,)),
            pltpu.SemaphoreType.REGULAR,
            pltpu.SemaphoreType.REGULAR,
            pltpu.SemaphoreType.REGULAR,
            pltpu.SemaphoreType.REGULAR,
        ],
        compiler_params=pltpu.CompilerParams(
            collective_id=0, vmem_limit_bytes=64 * 1024 * 1024),
    )(xb, wb, sp)


# device time: 275682 ns/iter; 1.0965x vs baseline; 1.0018x over previous
import jax
import jax.numpy as jnp
from jax import lax
from jax.experimental import pallas as pl
from jax.experimental.pallas import tpu as pltpu

N_PLANE = 8
N_Z = 4
M = 4096
N_COL = 2048
HC = N_COL // 2
PCH = M // N_PLANE
SUB = PCH // N_Z
HPC = PCH // 2


def kernel(x, w_mat, scale_x, scale_w):
    xb = x.astype(jnp.bfloat16)
    wb = w_mat.astype(jnp.bfloat16)
    sp = (scale_x.astype(jnp.float32) * scale_w.astype(jnp.float32)).reshape(1, 1)

    def body(x_ref, w_ref, sp_ref, out_ref,
             stageAR, stageAL, commAR, commAL,
             stageBR, stageBL, commBR, commBL,
             planeR, planeL, finalR, finalL,
             sendR, sendL, recvR, recvL,
             sendBR, sendBL, recvBR, recvBL,
             creditR, creditL, creditBR, creditBL):
        d = lax.axis_index("i")
        z = lax.div(d, N_PLANE)
        s_idx = lax.rem(d, N_PLANE)
        y_me = lax.div(s_idx, 2)
        x_me = lax.rem(s_idx + y_me, 2)
        c = jnp.where(x_me == 1, 1 + y_me, lax.rem(8 - y_me, 8))

        def plane_pos(cc):
            xx = jnp.where(cc == 0, 0, jnp.where(cc <= 4, 1, 0))
            yy = jnp.where(cc == 0, 0, jnp.where(cc <= 4, cc - 1, 8 - cc))
            ss = 2 * yy + lax.rem(xx + yy, 2)
            return z * N_PLANE + ss

        qr = plane_pos(lax.rem(c + 1, N_PLANE))
        ql = plane_pos(lax.rem(c + N_PLANE - 1, N_PLANE))
        zr = lax.rem(z + 1, N_Z) * N_PLANE + s_idx
        zl = lax.rem(z + N_Z - 1, N_Z) * N_PLANE + s_idx

        barrier = pltpu.get_barrier_semaphore()
        for nbr in (ql, qr):
            pl.semaphore_signal(barrier, inc=1, device_id=(nbr,),
                                device_id_type=pl.DeviceIdType.MESH)
        pl.semaphore_wait(barrier, 2)

        pl.semaphore_signal(creditBR, inc=2, device_id=(zl,),
                            device_id_type=pl.DeviceIdType.MESH)
        pl.semaphore_signal(creditBL, inc=2, device_id=(zr,),
                            device_id_type=pl.DeviceIdType.MESH)

        scale = sp_ref[0, 0]

        def pchunkA(j, dir_):
            xa = x_ref[pl.ds(j * PCH, PCH), :]
            wa = w_ref[:, dir_ * HC:(dir_ + 1) * HC]
            return lax.dot_general(
                xa, wa,
                dimension_numbers=(((1,), (0,)), ((), ())),
                preferred_element_type=jnp.float32).astype(jnp.bfloat16)

        def silu(acc):
            yv = acc * scale
            return yv / (1.0 + jnp.exp(-jnp.clip(yv, -60.0, 60.0)))

        def exchange(stR, cmR, stL, cmL, ssR, rsR, ssL, rsL, slot, tR, tL):
            rr = pltpu.make_async_remote_copy(
                src_ref=stR.at[slot], dst_ref=cmR.at[slot],
                send_sem=ssR.at[slot], recv_sem=rsR.at[slot],
                device_id=(tR,), device_id_type=pl.DeviceIdType.MESH)
            rl = pltpu.make_async_remote_copy(
                src_ref=stL.at[slot], dst_ref=cmL.at[slot],
                send_sem=ssL.at[slot], recv_sem=rsL.at[slot],
                device_id=(tL,), device_id_type=pl.DeviceIdType.MESH)
            rr.start()
            rl.start()
            return rr, rl

        def exchange_subs(slot):
            subs = []
            for u in (0, 1):
                rr = pltpu.make_async_remote_copy(
                    src_ref=stageAR.at[slot, pl.ds(u * HPC, HPC), :],
                    dst_ref=commAR.at[slot, pl.ds(u * HPC, HPC), :],
                    send_sem=sendR.at[slot * 2 + u],
                    recv_sem=recvR.at[slot * 2 + u],
                    device_id=(qr,), device_id_type=pl.DeviceIdType.MESH)
                rl = pltpu.make_async_remote_copy(
                    src_ref=stageAL.at[slot, pl.ds(u * HPC, HPC), :],
                    dst_ref=commAL.at[slot, pl.ds(u * HPC, HPC), :],
                    send_sem=sendL.at[slot * 2 + u],
                    recv_sem=recvL.at[slot * 2 + u],
                    device_id=(ql,), device_id_type=pl.DeviceIdType.MESH)
                subs.append((rr, rl))
            subs[0][0].start()
            subs[0][1].start()
            subs[1][0].start()
            subs[1][1].start()
            return subs

        stageAR[0, :, :] = pchunkA(c, 0)
        stageAL[0, :, :] = pchunkA(c, 1)

        for s in range(N_PLANE - 1):
            slot, nxt = s % 2, (s + 1) % 2
            if s >= 2:
                pl.semaphore_wait(creditR, 1)
                pl.semaphore_wait(creditL, 1)
            subs = exchange_subs(slot)
            jR = lax.rem(c - (s + 1) + 2 * N_PLANE, N_PLANE)
            jL = lax.rem(c + (s + 1), N_PLANE)
            pcR = pchunkA(jR, 0)
            pcL = pchunkA(jL, 1)
            for u in (0, 1):
                rr, rl = subs[u]
                rr.wait_send()
                rl.wait_send()
                rr.wait_recv()
                rl.wait_recv()
                rsl = slice(u * HPC, (u + 1) * HPC)
                accR = commAR[slot, rsl, :] + pcR[rsl, :]
                accL = commAL[slot, rsl, :] + pcL[rsl, :]
                if s < N_PLANE - 2:
                    stageAR[nxt, rsl, :] = accR
                    stageAL[nxt, rsl, :] = accL
                else:
                    planeR[rsl, :] = accR
                    planeL[rsl, :] = accL
            pl.semaphore_signal(creditR, inc=1, device_id=(ql,),
                                device_id_type=pl.DeviceIdType.MESH)
            pl.semaphore_signal(creditL, inc=1, device_id=(qr,),
                                device_id_type=pl.DeviceIdType.MESH)

        rows_R = lax.rem(c + 1, N_PLANE) * PCH
        rows_L = lax.rem(c + N_PLANE - 1, N_PLANE) * PCH

        stageBR[0, :, :] = planeR[pl.ds(z * SUB, SUB), :]
        stageBL[0, :, :] = planeL[pl.ds(z * SUB, SUB), :]

        for b in range(2 * N_Z - 2):
            slot, nxt = b % 2, (b + 1) % 2
            pl.semaphore_wait(creditBR, 1)
            pl.semaphore_wait(creditBL, 1)
            rr, rl = exchange(stageBR, commBR, stageBL, commBL,
                              sendBR, recvBR, sendBL, recvBL, slot, zr, zl)
            rr.wait_send()
            rl.wait_send()
            rr.wait_recv()
            rl.wait_recv()
            if b < N_Z - 1:
                mR = lax.rem(z - (b + 1) + 2 * N_Z, N_Z)
                mL = lax.rem(z + (b + 1), N_Z)
                accR = commBR[slot, :, :] + planeR[pl.ds(mR * SUB, SUB), :]
                accL = commBL[slot, :, :] + planeL[pl.ds(mL * SUB, SUB), :]
                if b == N_Z - 2:
                    yR = silu(accR.astype(jnp.float32))
                    yL = silu(accL.astype(jnp.float32))
                    out_ref[pl.ds(rows_R + mR * SUB, SUB), 0:HC] = yR
                    out_ref[pl.ds(rows_L + mL * SUB, SUB), HC:N_COL] = yL
                    finalR[pl.ds(mR * SUB, SUB), :] = yR.astype(jnp.bfloat16)
                    finalL[pl.ds(mL * SUB, SUB), :] = yL.astype(jnp.bfloat16)
                    stageBR[nxt, :, :] = yR.astype(jnp.bfloat16)
                    stageBL[nxt, :, :] = yL.astype(jnp.bfloat16)
                else:
                    stageBR[nxt, :, :] = accR
                    stageBL[nxt, :, :] = accL
            else:
                t = b - (N_Z - 1)
                mR = lax.rem(z - t + 2 * N_Z, N_Z)
                mL = lax.rem(z + t, N_Z)
                finalR[pl.ds(mR * SUB, SUB), :] = commBR[slot, :, :]
                finalL[pl.ds(mL * SUB, SUB), :] = commBL[slot, :, :]
                out_ref[pl.ds(rows_R + mR * SUB, SUB), 0:HC] = (
                    commBR[slot, :, :].astype(jnp.float32))
                out_ref[pl.ds(rows_L + mL * SUB, SUB), HC:N_COL] = (
                    commBL[slot, :, :].astype(jnp.float32))
                if b < 2 * N_Z - 3:
                    stageBR[nxt, :, :] = commBR[slot, :, :]
                    stageBL[nxt, :, :] = commBL[slot, :, :]
            if b <= 2 * N_Z - 5:
                pl.semaphore_signal(creditBR, inc=1, device_id=(zl,),
                                    device_id_type=pl.DeviceIdType.MESH)
                pl.semaphore_signal(creditBL, inc=1, device_id=(zr,),
                                    device_id_type=pl.DeviceIdType.MESH)

        stageAR[0, :, :] = finalR[:, :]
        stageAL[0, :, :] = finalL[:, :]

        for t in range(N_PLANE - 1):
            slot, nxt = t % 2, (t + 1) % 2
            if t == 0:
                pl.semaphore_wait(creditR, 2)
                pl.semaphore_wait(creditL, 2)
            elif t >= 2:
                pl.semaphore_wait(creditR, 1)
                pl.semaphore_wait(creditL, 1)
            subs = exchange_subs(slot)
            rowsRr = lax.rem(c - t + 2 * N_PLANE, N_PLANE) * PCH
            rowsLr = lax.rem(c + t, N_PLANE) * PCH
            for u in (0, 1):
                rr, rl = subs[u]
                rr.wait_send()
                rl.wait_send()
                rr.wait_recv()
                rl.wait_recv()
                rsl = slice(u * HPC, (u + 1) * HPC)
                out_ref[pl.ds(rowsRr + u * HPC, HPC), 0:HC] = (
                    commAR[slot, rsl, :].astype(jnp.float32))
                out_ref[pl.ds(rowsLr + u * HPC, HPC), HC:N_COL] = (
                    commAL[slot, rsl, :].astype(jnp.float32))
                if t < N_PLANE - 2:
                    stageAR[nxt, rsl, :] = commAR[slot, rsl, :]
                    stageAL[nxt, rsl, :] = commAL[slot, rsl, :]
            if t <= N_PLANE - 4:
                pl.semaphore_signal(creditR, inc=1, device_id=(ql,),
                                    device_id_type=pl.DeviceIdType.MESH)
                pl.semaphore_signal(creditL, inc=1, device_id=(qr,),
                                    device_id_type=pl.DeviceIdType.MESH)

    return pl.pallas_call(
        body,
        out_shape=jax.ShapeDtypeStruct((M, N_COL), jnp.float32),
        in_specs=[
            pl.BlockSpec(memory_space=pltpu.VMEM),
            pl.BlockSpec(memory_space=pltpu.VMEM),
            pl.BlockSpec(memory_space=pltpu.SMEM),
        ],
        out_specs=pl.BlockSpec(memory_space=pltpu.VMEM),
        scratch_shapes=[
            pltpu.VMEM((2, PCH, HC), jnp.bfloat16),
            pltpu.VMEM((2, PCH, HC), jnp.bfloat16),
            pltpu.VMEM((2, PCH, HC), jnp.bfloat16),
            pltpu.VMEM((2, PCH, HC), jnp.bfloat16),
            pltpu.VMEM((2, SUB, HC), jnp.bfloat16),
            pltpu.VMEM((2, SUB, HC), jnp.bfloat16),
            pltpu.VMEM((2, SUB, HC), jnp.bfloat16),
            pltpu.VMEM((2, SUB, HC), jnp.bfloat16),
            pltpu.VMEM((PCH, HC), jnp.bfloat16),
            pltpu.VMEM((PCH, HC), jnp.bfloat16),
            pltpu.VMEM((PCH, HC), jnp.bfloat16),
            pltpu.VMEM((PCH, HC), jnp.bfloat16),
            pltpu.SemaphoreType.DMA((4,)),
            pltpu.SemaphoreType.DMA((4,)),
            pltpu.SemaphoreType.DMA((4,)),
            pltpu.SemaphoreType.DMA((4,)),
            pltpu.SemaphoreType.DMA((2,)),
            pltpu.SemaphoreType.DMA((2,)),
            pltpu.SemaphoreType.DMA((2,)),
            pltpu.SemaphoreType.DMA((2,)),
            pltpu.SemaphoreType.REGULAR,
            pltpu.SemaphoreType.REGULAR,
            pltpu.SemaphoreType.REGULAR,
            pltpu.SemaphoreType.REGULAR,
        ],
        compiler_params=pltpu.CompilerParams(
            collective_id=0, vmem_limit_bytes=64 * 1024 * 1024),
    )(xb, wb, sp)
